# baseline (speedup 1.0000x reference)
import functools

import jax
import jax.numpy as jnp
from jax import lax
from jax.experimental import pallas as pl
from jax.experimental.pallas import tpu as pltpu
from jax.experimental.pallas import tpu_sc as plsc

NUM_TOKENS = 1000000
EMBED_DIM = 64
BATCH = 4096
SEQ = 200

NC = 2
NS = 16
NW = NC * NS

B_TOTAL = BATCH * SEQ
PER_W = B_TOTAL // NW
CHUNK = 128
N_STEPS = PER_W // CHUNK
N_PAIRS = N_STEPS // 2

W_ROWS = 40
W_CHUNKS = (NUM_TOKENS // 2) // W_ROWS
W_PAIRS = (W_CHUNKS + 2 * NW - 1) // (2 * NW)


@jax.jit
def _embed_gather(token_ids, emb_table):
    mesh = plsc.VectorSubcoreMesh(core_axis_name="c", subcore_axis_name="s")

    flat = token_ids.reshape(-1)
    hi_half = flat >= (NUM_TOKENS // 2)
    pidx3 = jnp.where(hi_half, flat - NUM_TOKENS // 2, flat).reshape(
        NW, N_STEPS, CHUNK
    )
    h3 = hi_half.astype(jnp.float32).reshape(NW, N_STEPS, CHUNK)

    @functools.partial(
        pl.kernel,
        mesh=mesh,
        out_type=jax.ShapeDtypeStruct(
            (NUM_TOKENS // 2, 2 * EMBED_DIM), jnp.float32
        ),
        scratch_types=[
            pltpu.VMEM((W_ROWS, EMBED_DIM), jnp.float32),
            pltpu.VMEM((W_ROWS, EMBED_DIM), jnp.float32),
            pltpu.VMEM((W_ROWS, EMBED_DIM), jnp.float32),
            pltpu.VMEM((W_ROWS, EMBED_DIM), jnp.float32),
            pltpu.VMEM((W_ROWS, 2 * EMBED_DIM), jnp.float32),
            pltpu.VMEM((W_ROWS, 2 * EMBED_DIM), jnp.float32),
            pltpu.SemaphoreType.DMA,
            pltpu.SemaphoreType.DMA,
            pltpu.SemaphoreType.DMA,
            pltpu.SemaphoreType.DMA,
        ],
    )
    def widen(tbl_hbm, wide_hbm, al0, ah0, al1, ah1, vb0, vb1,
              g0, g1, o0, o1):
        wid = lax.axis_index("s") * NC + lax.axis_index("c")
        HALF = NUM_TOKENS // 2

        def rd2(c, al, ah, sem):
            pltpu.async_copy(
                tbl_hbm.at[pl.ds(c * W_ROWS, W_ROWS)], al, sem
            )
            pltpu.async_copy(
                tbl_hbm.at[pl.ds(HALF + c * W_ROWS, W_ROWS)], ah, sem
            )

        def rd2_wait(c, al, ah, sem):
            pltpu.make_async_copy(
                tbl_hbm.at[pl.ds(c * W_ROWS, W_ROWS)], al, sem
            ).wait()
            pltpu.make_async_copy(
                tbl_hbm.at[pl.ds(HALF + c * W_ROWS, W_ROWS)], ah, sem
            ).wait()

        def pack(al, ah, vb):
            for r in range(W_ROWS):
                for seg in range(EMBED_DIM // 16):
                    vb[r, pl.ds(16 * seg, 16)] = al[r, pl.ds(16 * seg, 16)]
                    vb[r, pl.ds(EMBED_DIM + 16 * seg, 16)] = (
                        ah[r, pl.ds(16 * seg, 16)]
                    )

        def wr(c, vb, sem):
            pltpu.async_copy(
                vb, wide_hbm.at[pl.ds(c * W_ROWS, W_ROWS)], sem
            )

        def wr_wait(c, vb, sem):
            pltpu.make_async_copy(
                vb, wide_hbm.at[pl.ds(c * W_ROWS, W_ROWS)], sem
            ).wait()

        def body(i, _):
            c0 = wid + i * NW

            def do():
                rd2(c0, al0, ah0, g0)
                rd2_wait(c0, al0, ah0, g0)
                pack(al0, ah0, vb0)
                wr(c0, vb0, o0)
                wr_wait(c0, vb0, o0)

            pl.when(c0 < W_CHUNKS)(do)
            return 0

        lax.fori_loop(0, 2 * W_PAIRS, body, 0)

    tbl2 = widen(emb_table)

    @functools.partial(
        pl.kernel,
        mesh=mesh,
        out_type=jax.ShapeDtypeStruct((B_TOTAL, EMBED_DIM), jnp.float32),
        scratch_types=[
            pltpu.VMEM((N_STEPS, CHUNK), jnp.int32),
            pltpu.VMEM((N_STEPS, CHUNK), jnp.float32),
            pltpu.VMEM((CHUNK, 2 * EMBED_DIM), jnp.float32),
            pltpu.VMEM((CHUNK, 2 * EMBED_DIM), jnp.float32),
            pltpu.VMEM((CHUNK, EMBED_DIM), jnp.float32),
            pltpu.VMEM((CHUNK, EMBED_DIM), jnp.float32),
            pltpu.SemaphoreType.DMA,
            pltpu.SemaphoreType.DMA,
            pltpu.SemaphoreType.DMA,
            pltpu.SemaphoreType.DMA,
        ],
    )
    def k(pidx_hbm, h_hbm, tbl_hbm, out_hbm,
          pidx_v, h_v, buf0, buf1, rb0, rb1, g0, g1, w0, w1):
        wid = lax.axis_index("s") * NC + lax.axis_index("c")
        base = wid * PER_W
        pltpu.sync_copy(pidx_hbm.at[wid], pidx_v)
        pltpu.sync_copy(h_hbm.at[wid], h_v)

        def gather(j, buf, sem):
            pltpu.async_copy(tbl_hbm.at[pidx_v.at[j]], buf, sem)

        def gather_wait(j, buf, sem):
            pltpu.make_async_copy(tbl_hbm.at[pidx_v.at[j]], buf, sem).wait()

        def write(j, rb, sem):
            pltpu.async_copy(
                rb, out_hbm.at[pl.ds(base + j * CHUNK, CHUNK)], sem
            )

        def write_wait(j, rb, sem):
            pltpu.make_async_copy(
                rb, out_hbm.at[pl.ds(base + j * CHUNK, CHUNK)], sem
            ).wait()

        def select(j, buf, rb):
            for t0 in range(0, CHUNK, 16):
                hv = h_v[j, pl.ds(t0, 16)]
                for ti in range(16):
                    t = t0 + ti
                    sp = jnp.full((16,), ti, jnp.int32)
                    w = hv.at[sp].get(mode="promise_in_bounds")
                    wn = 1.0 - w
                    for seg in range(EMBED_DIM // 16):
                        lo = buf[t, pl.ds(16 * seg, 16)]
                        hi = buf[t, pl.ds(EMBED_DIM + 16 * seg, 16)]
                        rb[t, pl.ds(16 * seg, 16)] = lo * wn + hi * w

        gather(0, buf0, g0)

        def body(p, _):
            j0 = 2 * p
            gather_wait(j0, buf0, g0)
            gather(j0 + 1, buf1, g1)
            pl.when(p > 0)(lambda: write_wait(j0 - 2, rb0, w0))
            select(j0, buf0, rb0)
            write(j0, rb0, w0)

            gather_wait(j0 + 1, buf1, g1)
            pl.when(p + 1 < N_PAIRS)(lambda: gather(j0 + 2, buf0, g0))
            pl.when(p > 0)(lambda: write_wait(j0 - 1, rb1, w1))
            select(j0 + 1, buf1, rb1)
            write(j0 + 1, rb1, w1)
            return 0

        lax.fori_loop(0, N_PAIRS, body, 0)
        write_wait(N_STEPS - 2, rb0, w0)
        write_wait(N_STEPS - 1, rb1, w1)

    q = k(pidx3, h3, tbl2)
    return q.reshape(BATCH, SEQ, EMBED_DIM)


def kernel(token_ids, emb_table):
    return _embed_gather(token_ids, emb_table)


# speedup vs baseline: 1.2778x; 1.2778x over previous
import functools

import jax
import jax.numpy as jnp
from jax import lax
from jax.experimental import pallas as pl
from jax.experimental.pallas import tpu as pltpu
from jax.experimental.pallas import tpu_sc as plsc

NUM_TOKENS = 1000000
EMBED_DIM = 64
BATCH = 4096
SEQ = 200

NC = 2
NS = 16
NW = NC * NS

B_TOTAL = BATCH * SEQ
PER_W = B_TOTAL // NW
CHUNK = 128
N_STEPS = PER_W // CHUNK
N_PAIRS = N_STEPS // 2

W_ROWS = 40
W_CHUNKS = (NUM_TOKENS // 2) // W_ROWS
W_PAIRS = (W_CHUNKS + 2 * NW - 1) // (2 * NW)
W_ITERS = ((W_CHUNKS + NW - 1) // NW + 2) // 3


@jax.jit
def _embed_gather(token_ids, emb_table):
    mesh = plsc.VectorSubcoreMesh(core_axis_name="c", subcore_axis_name="s")

    flat = token_ids.reshape(-1)
    hi_half = flat >= (NUM_TOKENS // 2)
    pidx3 = jnp.where(hi_half, flat - NUM_TOKENS // 2, flat).reshape(
        NW, N_STEPS, CHUNK
    )
    h3 = hi_half.astype(jnp.float32).reshape(NW, N_STEPS, CHUNK)

    @functools.partial(
        pl.kernel,
        mesh=mesh,
        out_type=jax.ShapeDtypeStruct(
            (NUM_TOKENS // 2, 2 * EMBED_DIM), jnp.float32
        ),
        scratch_types=[
            pltpu.VMEM((W_ROWS, EMBED_DIM), jnp.float32),
            pltpu.VMEM((W_ROWS, EMBED_DIM), jnp.float32),
            pltpu.VMEM((W_ROWS, EMBED_DIM), jnp.float32),
            pltpu.VMEM((W_ROWS, EMBED_DIM), jnp.float32),
            pltpu.VMEM((W_ROWS, EMBED_DIM), jnp.float32),
            pltpu.VMEM((W_ROWS, EMBED_DIM), jnp.float32),
            pltpu.VMEM((W_ROWS, 2 * EMBED_DIM), jnp.float32),
            pltpu.VMEM((W_ROWS, 2 * EMBED_DIM), jnp.float32),
            pltpu.VMEM((W_ROWS, 2 * EMBED_DIM), jnp.float32),
            pltpu.SemaphoreType.DMA,
            pltpu.SemaphoreType.DMA,
            pltpu.SemaphoreType.DMA,
            pltpu.SemaphoreType.DMA,
            pltpu.SemaphoreType.DMA,
            pltpu.SemaphoreType.DMA,
        ],
    )
    def widen(tbl_hbm, wide_hbm, al0, ah0, al1, ah1, al2, ah2,
              vb0, vb1, vb2, g0, g1, g2, o0, o1, o2):
        wid = lax.axis_index("s") * NC + lax.axis_index("c")
        HALF = NUM_TOKENS // 2

        def rd2(c, al, ah, sem):
            pltpu.async_copy(
                tbl_hbm.at[pl.ds(c * W_ROWS, W_ROWS)], al, sem
            )
            pltpu.async_copy(
                tbl_hbm.at[pl.ds(HALF + c * W_ROWS, W_ROWS)], ah, sem
            )

        def rd2_wait(c, al, ah, sem):
            pltpu.make_async_copy(
                tbl_hbm.at[pl.ds(c * W_ROWS, W_ROWS)], al, sem
            ).wait()
            pltpu.make_async_copy(
                tbl_hbm.at[pl.ds(HALF + c * W_ROWS, W_ROWS)], ah, sem
            ).wait()

        def pack(al, ah, vb):
            for r in range(W_ROWS):
                for seg in range(EMBED_DIM // 16):
                    vb[r, pl.ds(16 * seg, 16)] = al[r, pl.ds(16 * seg, 16)]
                    vb[r, pl.ds(EMBED_DIM + 16 * seg, 16)] = (
                        ah[r, pl.ds(16 * seg, 16)]
                    )

        def wr(c, vb, sem):
            pltpu.async_copy(
                vb, wide_hbm.at[pl.ds(c * W_ROWS, W_ROWS)], sem
            )

        def wr_wait(c, vb, sem):
            pltpu.make_async_copy(
                vb, wide_hbm.at[pl.ds(c * W_ROWS, W_ROWS)], sem
            ).wait()

        slots = ((al0, ah0, vb0, g0, o0), (al1, ah1, vb1, g1, o1),
                 (al2, ah2, vb2, g2, o2))

        rd2(wid, al0, ah0, g0)
        rd2(wid + NW, al1, ah1, g1)

        def body(i, _):
            for j in range(3):
                al, ah, vb, g, o = slots[j]
                aln, ahn, _, gn, _ = slots[(j + 2) % 3]
                k = 3 * i + j
                c = wid + k * NW

                def stage(al=al, ah=ah, vb=vb, g=g, o=o, aln=aln, ahn=ahn,
                          gn=gn, k=k, c=c):
                    rd2_wait(c, al, ah, g)
                    pl.when(c + 2 * NW < W_CHUNKS)(
                        lambda: rd2(c + 2 * NW, aln, ahn, gn)
                    )
                    pl.when((k >= 3) & (c - 3 * NW < W_CHUNKS))(
                        lambda: wr_wait(c - 3 * NW, vb, o)
                    )
                    pack(al, ah, vb)
                    wr(c, vb, o)

                pl.when(c < W_CHUNKS)(stage)
            return 0

        lax.fori_loop(0, W_ITERS, body, 0)
        for j in range(3):
            _, _, vb, _, o = slots[j]
            c_a = wid + (3 * (W_ITERS - 1) + j) * NW
            c_b = c_a - 3 * NW
            pl.when(c_a < W_CHUNKS)(lambda vb=vb, o=o, c=c_a:
                                    wr_wait(c, vb, o))
            pl.when((c_a >= W_CHUNKS) & (c_b < W_CHUNKS))(
                lambda vb=vb, o=o, c=c_b: wr_wait(c, vb, o)
            )

    tbl2 = widen(emb_table)

    @functools.partial(
        pl.kernel,
        mesh=mesh,
        out_type=jax.ShapeDtypeStruct((B_TOTAL, EMBED_DIM), jnp.float32),
        scratch_types=[
            pltpu.VMEM((N_STEPS, CHUNK), jnp.int32),
            pltpu.VMEM((N_STEPS, CHUNK), jnp.float32),
            pltpu.VMEM((CHUNK, 2 * EMBED_DIM), jnp.float32),
            pltpu.VMEM((CHUNK, 2 * EMBED_DIM), jnp.float32),
            pltpu.VMEM((CHUNK, EMBED_DIM), jnp.float32),
            pltpu.VMEM((CHUNK, EMBED_DIM), jnp.float32),
            pltpu.SemaphoreType.DMA,
            pltpu.SemaphoreType.DMA,
            pltpu.SemaphoreType.DMA,
            pltpu.SemaphoreType.DMA,
        ],
    )
    def k(pidx_hbm, h_hbm, tbl_hbm, out_hbm,
          pidx_v, h_v, buf0, buf1, rb0, rb1, g0, g1, w0, w1):
        wid = lax.axis_index("s") * NC + lax.axis_index("c")
        base = wid * PER_W
        pltpu.sync_copy(pidx_hbm.at[wid], pidx_v)
        pltpu.sync_copy(h_hbm.at[wid], h_v)

        def gather(j, buf, sem):
            pltpu.async_copy(tbl_hbm.at[pidx_v.at[j]], buf, sem)

        def gather_wait(j, buf, sem):
            pltpu.make_async_copy(tbl_hbm.at[pidx_v.at[j]], buf, sem).wait()

        def write(j, rb, sem):
            pltpu.async_copy(
                rb, out_hbm.at[pl.ds(base + j * CHUNK, CHUNK)], sem
            )

        def write_wait(j, rb, sem):
            pltpu.make_async_copy(
                rb, out_hbm.at[pl.ds(base + j * CHUNK, CHUNK)], sem
            ).wait()

        def select(j, buf, rb):
            for t0 in range(0, CHUNK, 16):
                hv = h_v[j, pl.ds(t0, 16)]
                for ti in range(16):
                    t = t0 + ti
                    sp = jnp.full((16,), ti, jnp.int32)
                    w = hv.at[sp].get(mode="promise_in_bounds")
                    wn = 1.0 - w
                    for seg in range(EMBED_DIM // 16):
                        lo = buf[t, pl.ds(16 * seg, 16)]
                        hi = buf[t, pl.ds(EMBED_DIM + 16 * seg, 16)]
                        rb[t, pl.ds(16 * seg, 16)] = lo * wn + hi * w

        gather(0, buf0, g0)

        def body(p, _):
            j0 = 2 * p
            gather_wait(j0, buf0, g0)
            gather(j0 + 1, buf1, g1)
            pl.when(p > 0)(lambda: write_wait(j0 - 2, rb0, w0))
            select(j0, buf0, rb0)
            write(j0, rb0, w0)

            gather_wait(j0 + 1, buf1, g1)
            pl.when(p + 1 < N_PAIRS)(lambda: gather(j0 + 2, buf0, g0))
            pl.when(p > 0)(lambda: write_wait(j0 - 1, rb1, w1))
            select(j0 + 1, buf1, rb1)
            write(j0 + 1, rb1, w1)
            return 0

        lax.fori_loop(0, N_PAIRS, body, 0)
        write_wait(N_STEPS - 2, rb0, w0)
        write_wait(N_STEPS - 1, rb1, w1)

    q = k(pidx3, h3, tbl2)
    return q.reshape(BATCH, SEQ, EMBED_DIM)


def kernel(token_ids, emb_table):
    return _embed_gather(token_ids, emb_table)


# speedup vs baseline: 1.3004x; 1.0176x over previous
import functools

import jax
import jax.numpy as jnp
from jax import lax
from jax.experimental import pallas as pl
from jax.experimental.pallas import tpu as pltpu
from jax.experimental.pallas import tpu_sc as plsc

NUM_TOKENS = 1000000
EMBED_DIM = 64
BATCH = 4096
SEQ = 200

NC = 2
NS = 16
NW = NC * NS

B_TOTAL = BATCH * SEQ
PER_W = B_TOTAL // NW
CHUNK = 128
N_STEPS = PER_W // CHUNK
N_PAIRS = N_STEPS // 2

W_ROWS = 80
W_CHUNKS = (NUM_TOKENS // 2) // W_ROWS
W_PAIRS = (W_CHUNKS + 2 * NW - 1) // (2 * NW)
W_ITERS = ((W_CHUNKS + NW - 1) // NW + 2) // 3


@jax.jit
def _embed_gather(token_ids, emb_table):
    mesh = plsc.VectorSubcoreMesh(core_axis_name="c", subcore_axis_name="s")

    flat = token_ids.reshape(-1)
    hi_half = flat >= (NUM_TOKENS // 2)
    pidx3 = jnp.where(hi_half, flat - NUM_TOKENS // 2, flat).reshape(
        NW, N_STEPS, CHUNK
    )
    h3 = hi_half.astype(jnp.float32).reshape(NW, N_STEPS, CHUNK)

    @functools.partial(
        pl.kernel,
        mesh=mesh,
        out_type=jax.ShapeDtypeStruct(
            (NUM_TOKENS // 2, 2 * EMBED_DIM), jnp.float32
        ),
        scratch_types=[
            pltpu.VMEM((W_ROWS, EMBED_DIM), jnp.float32),
            pltpu.VMEM((W_ROWS, EMBED_DIM), jnp.float32),
            pltpu.VMEM((W_ROWS, EMBED_DIM), jnp.float32),
            pltpu.VMEM((W_ROWS, EMBED_DIM), jnp.float32),
            pltpu.VMEM((W_ROWS, EMBED_DIM), jnp.float32),
            pltpu.VMEM((W_ROWS, EMBED_DIM), jnp.float32),
            pltpu.VMEM((W_ROWS, 2 * EMBED_DIM), jnp.float32),
            pltpu.VMEM((W_ROWS, 2 * EMBED_DIM), jnp.float32),
            pltpu.VMEM((W_ROWS, 2 * EMBED_DIM), jnp.float32),
            pltpu.SemaphoreType.DMA,
            pltpu.SemaphoreType.DMA,
            pltpu.SemaphoreType.DMA,
            pltpu.SemaphoreType.DMA,
            pltpu.SemaphoreType.DMA,
            pltpu.SemaphoreType.DMA,
        ],
    )
    def widen(tbl_hbm, wide_hbm, al0, ah0, al1, ah1, al2, ah2,
              vb0, vb1, vb2, g0, g1, g2, o0, o1, o2):
        wid = lax.axis_index("s") * NC + lax.axis_index("c")
        HALF = NUM_TOKENS // 2

        def rd2(c, al, ah, sem):
            pltpu.async_copy(
                tbl_hbm.at[pl.ds(c * W_ROWS, W_ROWS)], al, sem
            )
            pltpu.async_copy(
                tbl_hbm.at[pl.ds(HALF + c * W_ROWS, W_ROWS)], ah, sem
            )

        def rd2_wait(c, al, ah, sem):
            pltpu.make_async_copy(
                tbl_hbm.at[pl.ds(c * W_ROWS, W_ROWS)], al, sem
            ).wait()
            pltpu.make_async_copy(
                tbl_hbm.at[pl.ds(HALF + c * W_ROWS, W_ROWS)], ah, sem
            ).wait()

        def pack(al, ah, vb):
            for r in range(W_ROWS):
                for seg in range(EMBED_DIM // 16):
                    vb[r, pl.ds(16 * seg, 16)] = al[r, pl.ds(16 * seg, 16)]
                    vb[r, pl.ds(EMBED_DIM + 16 * seg, 16)] = (
                        ah[r, pl.ds(16 * seg, 16)]
                    )

        def wr(c, vb, sem):
            pltpu.async_copy(
                vb, wide_hbm.at[pl.ds(c * W_ROWS, W_ROWS)], sem
            )

        def wr_wait(c, vb, sem):
            pltpu.make_async_copy(
                vb, wide_hbm.at[pl.ds(c * W_ROWS, W_ROWS)], sem
            ).wait()

        slots = ((al0, ah0, vb0, g0, o0), (al1, ah1, vb1, g1, o1),
                 (al2, ah2, vb2, g2, o2))

        rd2(wid, al0, ah0, g0)
        rd2(wid + NW, al1, ah1, g1)

        def body(i, _):
            for j in range(3):
                al, ah, vb, g, o = slots[j]
                aln, ahn, _, gn, _ = slots[(j + 2) % 3]
                k = 3 * i + j
                c = wid + k * NW

                def stage(al=al, ah=ah, vb=vb, g=g, o=o, aln=aln, ahn=ahn,
                          gn=gn, k=k, c=c):
                    rd2_wait(c, al, ah, g)
                    pl.when(c + 2 * NW < W_CHUNKS)(
                        lambda: rd2(c + 2 * NW, aln, ahn, gn)
                    )
                    pl.when((k >= 3) & (c - 3 * NW < W_CHUNKS))(
                        lambda: wr_wait(c - 3 * NW, vb, o)
                    )
                    pack(al, ah, vb)
                    wr(c, vb, o)

                pl.when(c < W_CHUNKS)(stage)
            return 0

        lax.fori_loop(0, W_ITERS, body, 0)
        for j in range(3):
            _, _, vb, _, o = slots[j]
            c_a = wid + (3 * (W_ITERS - 1) + j) * NW
            c_b = c_a - 3 * NW
            pl.when(c_a < W_CHUNKS)(lambda vb=vb, o=o, c=c_a:
                                    wr_wait(c, vb, o))
            pl.when((c_a >= W_CHUNKS) & (c_b < W_CHUNKS))(
                lambda vb=vb, o=o, c=c_b: wr_wait(c, vb, o)
            )

    tbl2 = widen(emb_table)

    @functools.partial(
        pl.kernel,
        mesh=mesh,
        out_type=jax.ShapeDtypeStruct((B_TOTAL, EMBED_DIM), jnp.float32),
        scratch_types=[
            pltpu.VMEM((N_STEPS, CHUNK), jnp.int32),
            pltpu.VMEM((N_STEPS, CHUNK), jnp.float32),
            pltpu.VMEM((CHUNK, 2 * EMBED_DIM), jnp.float32),
            pltpu.VMEM((CHUNK, 2 * EMBED_DIM), jnp.float32),
            pltpu.VMEM((CHUNK, EMBED_DIM), jnp.float32),
            pltpu.VMEM((CHUNK, EMBED_DIM), jnp.float32),
            pltpu.SemaphoreType.DMA,
            pltpu.SemaphoreType.DMA,
            pltpu.SemaphoreType.DMA,
            pltpu.SemaphoreType.DMA,
        ],
    )
    def k(pidx_hbm, h_hbm, tbl_hbm, out_hbm,
          pidx_v, h_v, buf0, buf1, rb0, rb1, g0, g1, w0, w1):
        wid = lax.axis_index("s") * NC + lax.axis_index("c")
        base = wid * PER_W
        pltpu.sync_copy(pidx_hbm.at[wid], pidx_v)
        pltpu.sync_copy(h_hbm.at[wid], h_v)

        def gather(j, buf, sem):
            pltpu.async_copy(tbl_hbm.at[pidx_v.at[j]], buf, sem)

        def gather_wait(j, buf, sem):
            pltpu.make_async_copy(tbl_hbm.at[pidx_v.at[j]], buf, sem).wait()

        def write(j, rb, sem):
            pltpu.async_copy(
                rb, out_hbm.at[pl.ds(base + j * CHUNK, CHUNK)], sem
            )

        def write_wait(j, rb, sem):
            pltpu.make_async_copy(
                rb, out_hbm.at[pl.ds(base + j * CHUNK, CHUNK)], sem
            ).wait()

        def select(j, buf, rb):
            for t0 in range(0, CHUNK, 16):
                hv = h_v[j, pl.ds(t0, 16)]
                for ti in range(16):
                    t = t0 + ti
                    sp = jnp.full((16,), ti, jnp.int32)
                    w = hv.at[sp].get(mode="promise_in_bounds")
                    wn = 1.0 - w
                    for seg in range(EMBED_DIM // 16):
                        lo = buf[t, pl.ds(16 * seg, 16)]
                        hi = buf[t, pl.ds(EMBED_DIM + 16 * seg, 16)]
                        rb[t, pl.ds(16 * seg, 16)] = lo * wn + hi * w

        gather(0, buf0, g0)

        def body(p, _):
            j0 = 2 * p
            gather_wait(j0, buf0, g0)
            gather(j0 + 1, buf1, g1)
            pl.when(p > 0)(lambda: write_wait(j0 - 2, rb0, w0))
            select(j0, buf0, rb0)
            write(j0, rb0, w0)

            gather_wait(j0 + 1, buf1, g1)
            pl.when(p + 1 < N_PAIRS)(lambda: gather(j0 + 2, buf0, g0))
            pl.when(p > 0)(lambda: write_wait(j0 - 1, rb1, w1))
            select(j0 + 1, buf1, rb1)
            write(j0 + 1, rb1, w1)
            return 0

        lax.fori_loop(0, N_PAIRS, body, 0)
        write_wait(N_STEPS - 2, rb0, w0)
        write_wait(N_STEPS - 1, rb1, w1)

    q = k(pidx3, h3, tbl2)
    return q.reshape(BATCH, SEQ, EMBED_DIM)


def kernel(token_ids, emb_table):
    return _embed_gather(token_ids, emb_table)


# speedup vs baseline: 1.3137x; 1.0103x over previous
import functools

import jax
import jax.numpy as jnp
from jax import lax
from jax.experimental import pallas as pl
from jax.experimental.pallas import tpu as pltpu
from jax.experimental.pallas import tpu_sc as plsc

NUM_TOKENS = 1000000
EMBED_DIM = 64
BATCH = 4096
SEQ = 200

NC = 2
NS = 16
NW = NC * NS

B_TOTAL = BATCH * SEQ
PER_W = B_TOTAL // NW
CHUNK = 128
N_STEPS = PER_W // CHUNK
N_PAIRS = N_STEPS // 2


@jax.jit
def _embed_gather(token_ids, emb_table):
    mesh = plsc.VectorSubcoreMesh(core_axis_name="c", subcore_axis_name="s")

    flat = token_ids.reshape(-1)
    pidx3 = jax.lax.shift_right_logical(flat, 1).reshape(NW, N_STEPS, CHUNK)
    h3 = jnp.bitwise_and(flat, 1).astype(jnp.float32).reshape(
        NW, N_STEPS, CHUNK
    )
    tbl2 = emb_table.reshape(NUM_TOKENS // 2, 2 * EMBED_DIM)

    @functools.partial(
        pl.kernel,
        mesh=mesh,
        out_type=jax.ShapeDtypeStruct((B_TOTAL, EMBED_DIM), jnp.float32),
        scratch_types=[
            pltpu.VMEM((N_STEPS, CHUNK), jnp.int32),
            pltpu.VMEM((N_STEPS, CHUNK), jnp.float32),
            pltpu.VMEM((CHUNK, 2 * EMBED_DIM), jnp.float32),
            pltpu.VMEM((CHUNK, 2 * EMBED_DIM), jnp.float32),
            pltpu.VMEM((CHUNK, EMBED_DIM), jnp.float32),
            pltpu.VMEM((CHUNK, EMBED_DIM), jnp.float32),
            pltpu.SemaphoreType.DMA,
            pltpu.SemaphoreType.DMA,
            pltpu.SemaphoreType.DMA,
            pltpu.SemaphoreType.DMA,
        ],
    )
    def k(pidx_hbm, h_hbm, tbl_hbm, out_hbm,
          pidx_v, h_v, buf0, buf1, rb0, rb1, g0, g1, w0, w1):
        wid = lax.axis_index("s") * NC + lax.axis_index("c")
        base = wid * PER_W
        pltpu.sync_copy(pidx_hbm.at[wid], pidx_v)
        pltpu.sync_copy(h_hbm.at[wid], h_v)

        def gather(j, buf, sem):
            pltpu.async_copy(tbl_hbm.at[pidx_v.at[j]], buf, sem)

        def gather_wait(j, buf, sem):
            pltpu.make_async_copy(tbl_hbm.at[pidx_v.at[j]], buf, sem).wait()

        def write(j, rb, sem):
            pltpu.async_copy(
                rb, out_hbm.at[pl.ds(base + j * CHUNK, CHUNK)], sem
            )

        def write_wait(j, rb, sem):
            pltpu.make_async_copy(
                rb, out_hbm.at[pl.ds(base + j * CHUNK, CHUNK)], sem
            ).wait()

        def select(j, buf, rb):
            for t0 in range(0, CHUNK, 16):
                hv = h_v[j, pl.ds(t0, 16)]
                for ti in range(16):
                    t = t0 + ti
                    sp = jnp.full((16,), ti, jnp.int32)
                    w = hv.at[sp].get(mode="promise_in_bounds")
                    wn = 1.0 - w
                    for seg in range(EMBED_DIM // 16):
                        lo = buf[t, pl.ds(16 * seg, 16)]
                        hi = buf[t, pl.ds(EMBED_DIM + 16 * seg, 16)]
                        rb[t, pl.ds(16 * seg, 16)] = lo * wn + hi * w

        gather(0, buf0, g0)

        def body(p, _):
            j0 = 2 * p
            gather_wait(j0, buf0, g0)
            gather(j0 + 1, buf1, g1)
            pl.when(p > 0)(lambda: write_wait(j0 - 2, rb0, w0))
            select(j0, buf0, rb0)
            write(j0, rb0, w0)

            gather_wait(j0 + 1, buf1, g1)
            pl.when(p + 1 < N_PAIRS)(lambda: gather(j0 + 2, buf0, g0))
            pl.when(p > 0)(lambda: write_wait(j0 - 1, rb1, w1))
            select(j0 + 1, buf1, rb1)
            write(j0 + 1, rb1, w1)
            return 0

        lax.fori_loop(0, N_PAIRS, body, 0)
        write_wait(N_STEPS - 2, rb0, w0)
        write_wait(N_STEPS - 1, rb1, w1)

    q = k(pidx3, h3, tbl2)
    return q.reshape(BATCH, SEQ, EMBED_DIM)


def kernel(token_ids, emb_table):
    return _embed_gather(token_ids, emb_table)
